# baseline (speedup 1.0000x reference)
import functools

import jax
import jax.numpy as jnp
from jax import lax
from jax.experimental import pallas as pl
from jax.experimental.pallas import tpu as pltpu
from jax.experimental.pallas import tpu_sc as plsc

DIM = 2048
NUM_EXPERTS = 64
TOP_K = 8
N_GROUPS = 8
GROUP_SIZE = NUM_EXPERTS // N_GROUPS
TOPK_GROUP = 4
ROUTED_SCALING_FACTOR = 2.5

NEG = -1e30

NC = 2
NS = 16
NW = NC * NS
LANES = 16


def _scores_body(T, w_ref, x_ref, b_ref, s_ref, sc_ref):
    logits = lax.dot_general(
        w_ref[:], x_ref[:], (((1,), (1,)), ((), ())),
        preferred_element_type=jnp.float32)
    s = jax.nn.sigmoid(logits)
    s_ref[:] = s
    sc_ref[:] = s + b_ref[:]


def _tc_scores(x, W_gate, bias):
    n = x.shape[0]
    T = 2048
    b2 = bias.reshape(NUM_EXPERTS, 1)
    return pl.pallas_call(
        functools.partial(_scores_body, T),
        grid=(n // T,),
        in_specs=[
            pl.BlockSpec((NUM_EXPERTS, DIM), lambda i: (0, 0)),
            pl.BlockSpec((T, DIM), lambda i: (i, 0)),
            pl.BlockSpec((NUM_EXPERTS, 1), lambda i: (0, 0)),
        ],
        out_specs=[
            pl.BlockSpec((NUM_EXPERTS, T), lambda i: (0, i)),
            pl.BlockSpec((NUM_EXPERTS, T), lambda i: (0, i)),
        ],
        out_shape=[
            jax.ShapeDtypeStruct((NUM_EXPERTS, n), jnp.float32),
            jax.ShapeDtypeStruct((NUM_EXPERTS, n), jnp.float32),
        ],
    )(W_gate, x, b2)


def _tree_max(vs):
    while len(vs) > 1:
        vs = [jnp.maximum(vs[i], vs[i + 1]) for i in range(0, len(vs) - 1, 2)] \
            + ([vs[-1]] if len(vs) % 2 else [])
    return vs[0]


def _sc_router_body(tpw, s_hbm, sc_hbm, idx_hbm, w_hbm,
                    s_v, sc_v, idx_t, w_t, tmp_v):
    wid = lax.axis_index("s") * NC + lax.axis_index("c")
    base = wid * tpw
    pltpu.sync_copy(s_hbm.at[:, pl.ds(base, tpw)], s_v)
    pltpu.sync_copy(sc_hbm.at[:, pl.ds(base, tpw)], sc_v)
    negv = jnp.full((LANES,), NEG, jnp.float32)
    zf = jnp.zeros((LANES,), jnp.float32)

    def chunk_body(c, carry):
        col = c * LANES
        cs = pl.ds(col, LANES)
        gs = []
        for g in range(N_GROUPS):
            m1 = sc_v[g * GROUP_SIZE, cs]
            m2 = negv
            for j in range(1, GROUP_SIZE):
                v = sc_v[g * GROUP_SIZE + j, cs]
                hi = jnp.maximum(m1, v)
                m2 = jnp.maximum(m2, jnp.minimum(m1, v))
                m1 = hi
            gs.append(m1 + m2)
        onev = jnp.ones((LANES,), jnp.float32)
        allowed = [zf for _ in range(N_GROUPS)]
        for _ in range(TOPK_GROUP):
            m = _tree_max(gs)
            nf = onev
            for g in range(N_GROUPS):
                take = jnp.where(gs[g] == m, nf, zf)
                nf = nf - take
                allowed[g] = allowed[g] + take
                gs[g] = jnp.where(take > 0.0, negv, gs[g])
        for g in range(N_GROUPS):
            ok = allowed[g] > 0.0
            for j in range(GROUP_SIZE):
                e = g * GROUP_SIZE + j
                tmp_v[e, :] = jnp.where(ok, sc_v[e, cs], zf)

        idxs, ws = [], []
        for _ in range(TOP_K):
            vs = [tmp_v[e, :] for e in range(NUM_EXPERTS)]
            m = _tree_max(vs)
            nf = onev
            idxk = jnp.zeros((LANES,), jnp.int32)
            wv = zf
            for e in range(NUM_EXPERTS):
                v = tmp_v[e, :]
                take = jnp.where(v == m, nf, zf)
                nf = nf - take
                hit = take > 0.0
                idxk = jnp.where(hit, jnp.full((LANES,), e, jnp.int32), idxk)
                wv = jnp.where(hit, s_v[e, cs], wv)
                tmp_v[e, :] = jnp.where(hit, negv, v)
            idxs.append(idxk)
            ws.append(wv)

        denom = ws[0]
        for k in range(1, TOP_K):
            denom = denom + ws[k]
        scale = ROUTED_SCALING_FACTOR / (denom + 1e-20)
        for k in range(TOP_K):
            idx_t[k, cs] = idxs[k]
            w_t[k, cs] = ws[k] * scale
        return carry

    lax.fori_loop(0, tpw // LANES, chunk_body, 0)
    pltpu.sync_copy(idx_t, idx_hbm.at[:, pl.ds(base, tpw)])
    pltpu.sync_copy(w_t, w_hbm.at[:, pl.ds(base, tpw)])


def _sc_router(s, sc):
    n = s.shape[1]
    tpw = n // NW
    f = functools.partial(
        pl.kernel,
        out_type=[
            jax.ShapeDtypeStruct((TOP_K, n), jnp.int32),
            jax.ShapeDtypeStruct((TOP_K, n), jnp.float32),
        ],
        mesh=plsc.VectorSubcoreMesh(core_axis_name="c", subcore_axis_name="s"),
        scratch_types=[
            pltpu.VMEM((NUM_EXPERTS, tpw), jnp.float32),
            pltpu.VMEM((NUM_EXPERTS, tpw), jnp.float32),
            pltpu.VMEM((TOP_K, tpw), jnp.int32),
            pltpu.VMEM((TOP_K, tpw), jnp.float32),
            pltpu.VMEM((NUM_EXPERTS, LANES), jnp.float32),
        ],
    )(functools.partial(_sc_router_body, tpw))
    return f(s, sc)


def kernel(x, W_gate, e_score_correction_bias):
    s, sc = _tc_scores(x, W_gate, e_score_correction_bias)
    idx_t, w_t = _sc_router(s, sc)
    return idx_t.T, w_t.T


# speedup vs baseline: 1.1564x; 1.1564x over previous
import functools

import jax
import jax.numpy as jnp
from jax import lax
from jax.experimental import pallas as pl
from jax.experimental.pallas import tpu as pltpu
from jax.experimental.pallas import tpu_sc as plsc

DIM = 2048
NUM_EXPERTS = 64
TOP_K = 8
N_GROUPS = 8
GROUP_SIZE = NUM_EXPERTS // N_GROUPS
TOPK_GROUP = 4
ROUTED_SCALING_FACTOR = 2.5

NEG = -1e30

NC = 2
NS = 16
NW = NC * NS
LANES = 16


def _scores_body(T, w_ref, x_ref, b_ref, s_ref, sc_ref):
    logits = lax.dot_general(
        w_ref[:], x_ref[:], (((1,), (1,)), ((), ())),
        preferred_element_type=jnp.float32)
    s = jax.nn.sigmoid(logits)
    s_ref[:] = s
    sc_ref[:] = s + b_ref[:]


def _tc_scores(x, W_gate, bias):
    n = x.shape[0]
    T = 2048
    b2 = bias.reshape(NUM_EXPERTS, 1)
    return pl.pallas_call(
        functools.partial(_scores_body, T),
        grid=(n // T,),
        in_specs=[
            pl.BlockSpec((NUM_EXPERTS, DIM), lambda i: (0, 0)),
            pl.BlockSpec((T, DIM), lambda i: (i, 0)),
            pl.BlockSpec((NUM_EXPERTS, 1), lambda i: (0, 0)),
        ],
        out_specs=[
            pl.BlockSpec((NUM_EXPERTS, T), lambda i: (0, i)),
            pl.BlockSpec((NUM_EXPERTS, T), lambda i: (0, i)),
        ],
        out_shape=[
            jax.ShapeDtypeStruct((NUM_EXPERTS, n), jnp.float32),
            jax.ShapeDtypeStruct((NUM_EXPERTS, n), jnp.float32),
        ],
    )(W_gate, x, b2)


def _tree_max(vs):
    while len(vs) > 1:
        vs = [jnp.maximum(vs[i], vs[i + 1]) for i in range(0, len(vs) - 1, 2)] \
            + ([vs[-1]] if len(vs) % 2 else [])
    return vs[0]


def _top2_tree(vals):
    if len(vals) == 2:
        return jnp.maximum(vals[0], vals[1]), jnp.minimum(vals[0], vals[1])
    mid = len(vals) // 2
    a1, a2 = _top2_tree(vals[:mid])
    b1, b2 = _top2_tree(vals[mid:])
    return (jnp.maximum(a1, b1),
            jnp.maximum(jnp.minimum(a1, b1), jnp.maximum(a2, b2)))


def _argmin_pair_tree(pairs):
    while len(pairs) > 1:
        nxt = []
        for i in range(0, len(pairs) - 1, 2):
            ia, sa = pairs[i]
            ib, sb = pairs[i + 1]
            c = ia <= ib
            nxt.append((jnp.where(c, ia, ib), jnp.where(c, sa, sb)))
        if len(pairs) % 2:
            nxt.append(pairs[-1])
        pairs = nxt
    return pairs[0]


def _sc_router_body(tpw, s_hbm, sc_hbm, idx_hbm, w_hbm,
                    s_v, sc_v, idx_t, w_t, tmp_v):
    wid = lax.axis_index("s") * NC + lax.axis_index("c")
    base = wid * tpw
    pltpu.sync_copy(s_hbm.at[:, pl.ds(base, tpw)], s_v)
    pltpu.sync_copy(sc_hbm.at[:, pl.ds(base, tpw)], sc_v)
    negv = jnp.full((LANES,), NEG, jnp.float32)
    zf = jnp.zeros((LANES,), jnp.float32)

    def chunk_body(c, carry):
        col = c * LANES
        cs = pl.ds(col, LANES)
        gs = []
        for g in range(N_GROUPS):
            vals = [sc_v[g * GROUP_SIZE + j, cs] for j in range(GROUP_SIZE)]
            m1, m2 = _top2_tree(vals)
            gs.append(m1 + m2)
        onev = jnp.ones((LANES,), jnp.float32)
        big_g = jnp.full((LANES,), N_GROUPS, jnp.int32)
        allowed = [zf for _ in range(N_GROUPS)]
        for _ in range(TOPK_GROUP):
            m = _tree_max(gs)
            cands = [(jnp.where(gs[g] == m,
                                jnp.full((LANES,), g, jnp.int32), big_g), zf)
                     for g in range(N_GROUPS)]
            gidx, _ = _argmin_pair_tree(cands)
            for g in range(N_GROUPS):
                hit = gidx == g
                allowed[g] = allowed[g] + jnp.where(hit, onev, zf)
                gs[g] = jnp.where(hit, negv, gs[g])
        for g in range(N_GROUPS):
            ok = allowed[g] > 0.0
            for j in range(GROUP_SIZE):
                e = g * GROUP_SIZE + j
                tmp_v[e, :] = jnp.where(ok, sc_v[e, cs], zf)

        big_e = jnp.full((LANES,), NUM_EXPERTS, jnp.int32)
        idxs, ws = [], []
        for _ in range(TOP_K):
            vs = [tmp_v[e, :] for e in range(NUM_EXPERTS)]
            m = _tree_max(vs)
            cands = [(jnp.where(vs[e] == m,
                                jnp.full((LANES,), e, jnp.int32), big_e),
                      s_v[e, cs])
                     for e in range(NUM_EXPERTS)]
            idxk, wv = _argmin_pair_tree(cands)
            for e in range(NUM_EXPERTS):
                tmp_v[e, :] = jnp.where(idxk == e, negv, vs[e])
            idxs.append(idxk)
            ws.append(wv)

        denom = ws[0]
        for k in range(1, TOP_K):
            denom = denom + ws[k]
        scale = ROUTED_SCALING_FACTOR / (denom + 1e-20)
        for k in range(TOP_K):
            idx_t[k, cs] = idxs[k]
            w_t[k, cs] = ws[k] * scale
        return carry

    lax.fori_loop(0, tpw // LANES, chunk_body, 0)
    pltpu.sync_copy(idx_t, idx_hbm.at[:, pl.ds(base, tpw)])
    pltpu.sync_copy(w_t, w_hbm.at[:, pl.ds(base, tpw)])


def _sc_router(s, sc):
    n = s.shape[1]
    tpw = n // NW
    f = functools.partial(
        pl.kernel,
        out_type=[
            jax.ShapeDtypeStruct((TOP_K, n), jnp.int32),
            jax.ShapeDtypeStruct((TOP_K, n), jnp.float32),
        ],
        mesh=plsc.VectorSubcoreMesh(core_axis_name="c", subcore_axis_name="s"),
        scratch_types=[
            pltpu.VMEM((NUM_EXPERTS, tpw), jnp.float32),
            pltpu.VMEM((NUM_EXPERTS, tpw), jnp.float32),
            pltpu.VMEM((TOP_K, tpw), jnp.int32),
            pltpu.VMEM((TOP_K, tpw), jnp.float32),
            pltpu.VMEM((NUM_EXPERTS, LANES), jnp.float32),
        ],
    )(functools.partial(_sc_router_body, tpw))
    return f(s, sc)


def kernel(x, W_gate, e_score_correction_bias):
    s, sc = _tc_scores(x, W_gate, e_score_correction_bias)
    idx_t, w_t = _sc_router(s, sc)
    return idx_t.T, w_t.T


# speedup vs baseline: 2.1812x; 1.8861x over previous
import functools

import jax
import jax.numpy as jnp
from jax import lax
from jax.experimental import pallas as pl
from jax.experimental.pallas import tpu as pltpu
from jax.experimental.pallas import tpu_sc as plsc

DIM = 2048
NUM_EXPERTS = 64
TOP_K = 8
N_GROUPS = 8
GROUP_SIZE = NUM_EXPERTS // N_GROUPS
TOPK_GROUP = 4
ROUTED_SCALING_FACTOR = 2.5

NEG = -1e30

NC = 2
NS = 16
NW = NC * NS
LANES = 16


def _scores_body(T, w_ref, x_ref, b_ref, s_ref, sc_ref):
    logits = lax.dot_general(
        w_ref[:], x_ref[:], (((1,), (1,)), ((), ())),
        preferred_element_type=jnp.float32)
    s = jax.nn.sigmoid(logits)
    s_ref[:] = s
    sc_ref[:] = s + b_ref[:]


def _tc_scores(x, W_gate, bias):
    n = x.shape[0]
    T = 2048
    b2 = bias.reshape(NUM_EXPERTS, 1)
    return pl.pallas_call(
        functools.partial(_scores_body, T),
        grid=(n // T,),
        in_specs=[
            pl.BlockSpec((NUM_EXPERTS, DIM), lambda i: (0, 0)),
            pl.BlockSpec((T, DIM), lambda i: (i, 0)),
            pl.BlockSpec((NUM_EXPERTS, 1), lambda i: (0, 0)),
        ],
        out_specs=[
            pl.BlockSpec((NUM_EXPERTS, T), lambda i: (0, i)),
            pl.BlockSpec((NUM_EXPERTS, T), lambda i: (0, i)),
        ],
        out_shape=[
            jax.ShapeDtypeStruct((NUM_EXPERTS, n), jnp.float32),
            jax.ShapeDtypeStruct((NUM_EXPERTS, n), jnp.float32),
        ],
    )(W_gate, x, b2)


def _tree_max(vs):
    while len(vs) > 1:
        vs = [jnp.maximum(vs[i], vs[i + 1]) for i in range(0, len(vs) - 1, 2)] \
            + ([vs[-1]] if len(vs) % 2 else [])
    return vs[0]


def _top2_tree(vals):
    if len(vals) == 2:
        return jnp.maximum(vals[0], vals[1]), jnp.minimum(vals[0], vals[1])
    mid = len(vals) // 2
    a1, a2 = _top2_tree(vals[:mid])
    b1, b2 = _top2_tree(vals[mid:])
    return (jnp.maximum(a1, b1),
            jnp.maximum(jnp.minimum(a1, b1), jnp.maximum(a2, b2)))


def _argmin_pair_tree(pairs):
    while len(pairs) > 1:
        nxt = []
        for i in range(0, len(pairs) - 1, 2):
            ia, sa = pairs[i]
            ib, sb = pairs[i + 1]
            c = ia <= ib
            nxt.append((jnp.where(c, ia, ib), jnp.where(c, sa, sb)))
        if len(pairs) % 2:
            nxt.append(pairs[-1])
        pairs = nxt
    return pairs[0]


def _sc_router_body(tpw, s_hbm, sc_hbm, idx_hbm, w_hbm,
                    s_v, sc_v, idx_t, w_t, tmp_v):
    wid = lax.axis_index("s") * NC + lax.axis_index("c")
    base = wid * tpw
    pltpu.sync_copy(s_hbm.at[:, pl.ds(base, tpw)], s_v)
    pltpu.sync_copy(sc_hbm.at[:, pl.ds(base, tpw)], sc_v)
    negv = jnp.full((LANES,), NEG, jnp.float32)
    zf = jnp.zeros((LANES,), jnp.float32)

    def chunk_body(c, carry):
        col = c * LANES
        cs = pl.ds(col, LANES)
        gs = []
        for g in range(N_GROUPS):
            vals = [sc_v[g * GROUP_SIZE + j, cs] for j in range(GROUP_SIZE)]
            m1, m2 = _top2_tree(vals)
            gs.append(m1 + m2)
        onev = jnp.ones((LANES,), jnp.float32)
        big_g = jnp.full((LANES,), N_GROUPS, jnp.int32)
        allowed = [zf for _ in range(N_GROUPS)]
        for _ in range(TOPK_GROUP):
            m = _tree_max(gs)
            cands = [(jnp.where(gs[g] == m,
                                jnp.full((LANES,), g, jnp.int32), big_g), zf)
                     for g in range(N_GROUPS)]
            gidx, _ = _argmin_pair_tree(cands)
            for g in range(N_GROUPS):
                hit = gidx == g
                allowed[g] = allowed[g] + jnp.where(hit, onev, zf)
                gs[g] = jnp.where(hit, negv, gs[g])
        for g in range(N_GROUPS):
            ok = allowed[g] > 0.0
            for j in range(GROUP_SIZE):
                e = g * GROUP_SIZE + j
                tmp_v[e, :] = jnp.where(ok, sc_v[e, cs], zf)

        big_e = jnp.full((LANES,), NUM_EXPERTS, jnp.int32)

        def round_body(k, denom):
            vs = [tmp_v[e, :] for e in range(NUM_EXPERTS)]
            m = _tree_max(vs)
            cands = [(jnp.where(vs[e] == m,
                                jnp.full((LANES,), e, jnp.int32), big_e),
                      s_v[e, cs])
                     for e in range(NUM_EXPERTS)]
            idxk, wv = _argmin_pair_tree(cands)
            for e in range(NUM_EXPERTS):
                tmp_v[e, :] = jnp.where(idxk == e, negv, vs[e])
            idx_t[k, cs] = idxk
            w_t[k, cs] = wv
            return denom + wv

        denom = lax.fori_loop(0, TOP_K, round_body, zf)
        scale = ROUTED_SCALING_FACTOR / (denom + 1e-20)
        for k in range(TOP_K):
            w_t[k, cs] = w_t[k, cs] * scale
        return carry

    lax.fori_loop(0, tpw // LANES, chunk_body, 0)
    pltpu.sync_copy(idx_t, idx_hbm.at[:, pl.ds(base, tpw)])
    pltpu.sync_copy(w_t, w_hbm.at[:, pl.ds(base, tpw)])


def _sc_router(s, sc):
    n = s.shape[1]
    tpw = n // NW
    f = functools.partial(
        pl.kernel,
        out_type=[
            jax.ShapeDtypeStruct((TOP_K, n), jnp.int32),
            jax.ShapeDtypeStruct((TOP_K, n), jnp.float32),
        ],
        mesh=plsc.VectorSubcoreMesh(core_axis_name="c", subcore_axis_name="s"),
        scratch_types=[
            pltpu.VMEM((NUM_EXPERTS, tpw), jnp.float32),
            pltpu.VMEM((NUM_EXPERTS, tpw), jnp.float32),
            pltpu.VMEM((TOP_K, tpw), jnp.int32),
            pltpu.VMEM((TOP_K, tpw), jnp.float32),
            pltpu.VMEM((NUM_EXPERTS, LANES), jnp.float32),
        ],
    )(functools.partial(_sc_router_body, tpw))
    return f(s, sc)


def kernel(x, W_gate, e_score_correction_bias):
    s, sc = _tc_scores(x, W_gate, e_score_correction_bias)
    idx_t, w_t = _sc_router(s, sc)
    return idx_t.T, w_t.T


# speedup vs baseline: 2.1905x; 1.0043x over previous
import functools

import jax
import jax.numpy as jnp
from jax import lax
from jax.experimental import pallas as pl
from jax.experimental.pallas import tpu as pltpu
from jax.experimental.pallas import tpu_sc as plsc

DIM = 2048
NUM_EXPERTS = 64
TOP_K = 8
N_GROUPS = 8
GROUP_SIZE = NUM_EXPERTS // N_GROUPS
TOPK_GROUP = 4
ROUTED_SCALING_FACTOR = 2.5

NEG = -1e30

NC = 2
NS = 16
NW = NC * NS
LANES = 16


def _scores_body(T, w_ref, x_ref, b_ref, s_ref, sc_ref):
    logits = lax.dot_general(
        w_ref[:], x_ref[:], (((1,), (1,)), ((), ())),
        preferred_element_type=jnp.float32)
    s = jax.nn.sigmoid(logits)
    s_ref[:] = s
    sc_ref[:] = s + b_ref[:]


def _tc_scores(x, W_gate, bias):
    n = x.shape[0]
    T = 2048
    b2 = bias.reshape(NUM_EXPERTS, 1)
    return pl.pallas_call(
        functools.partial(_scores_body, T),
        grid=(n // T,),
        in_specs=[
            pl.BlockSpec((NUM_EXPERTS, DIM), lambda i: (0, 0)),
            pl.BlockSpec((T, DIM), lambda i: (i, 0)),
            pl.BlockSpec((NUM_EXPERTS, 1), lambda i: (0, 0)),
        ],
        out_specs=[
            pl.BlockSpec((NUM_EXPERTS, T), lambda i: (0, i)),
            pl.BlockSpec((NUM_EXPERTS, T), lambda i: (0, i)),
        ],
        out_shape=[
            jax.ShapeDtypeStruct((NUM_EXPERTS, n), jnp.float32),
            jax.ShapeDtypeStruct((NUM_EXPERTS, n), jnp.float32),
        ],
    )(W_gate, x, b2)


def _tree_max(vs):
    while len(vs) > 1:
        vs = [jnp.maximum(vs[i], vs[i + 1]) for i in range(0, len(vs) - 1, 2)] \
            + ([vs[-1]] if len(vs) % 2 else [])
    return vs[0]


def _top2_tree(vals):
    if len(vals) == 2:
        return jnp.maximum(vals[0], vals[1]), jnp.minimum(vals[0], vals[1])
    mid = len(vals) // 2
    a1, a2 = _top2_tree(vals[:mid])
    b1, b2 = _top2_tree(vals[mid:])
    return (jnp.maximum(a1, b1),
            jnp.maximum(jnp.minimum(a1, b1), jnp.maximum(a2, b2)))


def _argmin_pair_tree(pairs):
    while len(pairs) > 1:
        nxt = []
        for i in range(0, len(pairs) - 1, 2):
            ia, sa = pairs[i]
            ib, sb = pairs[i + 1]
            c = ia <= ib
            nxt.append((jnp.where(c, ia, ib), jnp.where(c, sa, sb)))
        if len(pairs) % 2:
            nxt.append(pairs[-1])
        pairs = nxt
    return pairs[0]


def _sc_router_body(tpw, s_hbm, sc_hbm, idx_hbm, w_hbm,
                    s_v, sc_v, idx_t, w_t, tmp_v):
    wid = lax.axis_index("s") * NC + lax.axis_index("c")
    base = wid * tpw
    pltpu.sync_copy(s_hbm.at[:, pl.ds(base, tpw)], s_v)
    pltpu.sync_copy(sc_hbm.at[:, pl.ds(base, tpw)], sc_v)
    negv = jnp.full((LANES,), NEG, jnp.float32)
    zf = jnp.zeros((LANES,), jnp.float32)

    def chunk_body(c, carry):
        col = c * LANES
        cs = pl.ds(col, LANES)
        gs = []
        for g in range(N_GROUPS):
            vals = [sc_v[g * GROUP_SIZE + j, cs] for j in range(GROUP_SIZE)]
            m1, m2 = _top2_tree(vals)
            gs.append(m1 + m2)
        onev = jnp.ones((LANES,), jnp.float32)
        big_g = jnp.full((LANES,), N_GROUPS, jnp.int32)
        allowed = [zf for _ in range(N_GROUPS)]
        for _ in range(TOPK_GROUP):
            m = _tree_max(gs)
            cands = [(jnp.where(gs[g] == m,
                                jnp.full((LANES,), g, jnp.int32), big_g), zf)
                     for g in range(N_GROUPS)]
            gidx, _ = _argmin_pair_tree(cands)
            for g in range(N_GROUPS):
                hit = gidx == g
                allowed[g] = allowed[g] + jnp.where(hit, onev, zf)
                gs[g] = jnp.where(hit, negv, gs[g])
        for g in range(N_GROUPS):
            ok = allowed[g] > 0.0
            for j in range(GROUP_SIZE):
                e = g * GROUP_SIZE + j
                tmp_v[e, :] = jnp.where(ok, sc_v[e, cs], zf)

        big_e = jnp.full((LANES,), NUM_EXPERTS, jnp.int32)

        def round_body(k, denom):
            m = _tree_max([tmp_v[e, :] for e in range(NUM_EXPERTS)])
            cands = [(jnp.where(tmp_v[e, :] == m,
                                jnp.full((LANES,), e, jnp.int32), big_e),
                      s_v[e, cs])
                     for e in range(NUM_EXPERTS)]
            idxk, wv = _argmin_pair_tree(cands)
            for e in range(NUM_EXPERTS):
                tmp_v[e, :] = jnp.where(idxk == e, negv, tmp_v[e, :])
            idx_t[k, cs] = idxk
            w_t[k, cs] = wv
            return denom + wv

        denom = lax.fori_loop(0, TOP_K, round_body, zf)
        scale = ROUTED_SCALING_FACTOR / (denom + 1e-20)
        for k in range(TOP_K):
            w_t[k, cs] = w_t[k, cs] * scale
        return carry

    lax.fori_loop(0, tpw // LANES, chunk_body, 0)
    pltpu.sync_copy(idx_t, idx_hbm.at[:, pl.ds(base, tpw)])
    pltpu.sync_copy(w_t, w_hbm.at[:, pl.ds(base, tpw)])


def _sc_router(s, sc):
    n = s.shape[1]
    tpw = n // NW
    f = functools.partial(
        pl.kernel,
        out_type=[
            jax.ShapeDtypeStruct((TOP_K, n), jnp.int32),
            jax.ShapeDtypeStruct((TOP_K, n), jnp.float32),
        ],
        mesh=plsc.VectorSubcoreMesh(core_axis_name="c", subcore_axis_name="s"),
        scratch_types=[
            pltpu.VMEM((NUM_EXPERTS, tpw), jnp.float32),
            pltpu.VMEM((NUM_EXPERTS, tpw), jnp.float32),
            pltpu.VMEM((TOP_K, tpw), jnp.int32),
            pltpu.VMEM((TOP_K, tpw), jnp.float32),
            pltpu.VMEM((NUM_EXPERTS, LANES), jnp.float32),
        ],
    )(functools.partial(_sc_router_body, tpw))
    return f(s, sc)


def kernel(x, W_gate, e_score_correction_bias):
    s, sc = _tc_scores(x, W_gate, e_score_correction_bias)
    idx_t, w_t = _sc_router(s, sc)
    return idx_t.T, w_t.T


# speedup vs baseline: 3.8973x; 1.7791x over previous
import functools

import jax
import jax.numpy as jnp
from jax.experimental import pallas as pl
from jax.experimental.pallas import tpu as pltpu

DIM = 2048
NUM_EXPERTS = 64
TOP_K = 8
N_GROUPS = 8
GROUP_SIZE = NUM_EXPERTS // N_GROUPS
TOPK_GROUP = 4
ROUTED_SCALING_FACTOR = 2.5

NEG = -1e30


def _router_body(T, w_ref, x_ref, b_ref, idx_ref, wt_ref, acc_ref):
    k = pl.program_id(1)
    part = jax.lax.dot_general(
        w_ref[:], x_ref[:], (((1,), (1,)), ((), ())),
        preferred_element_type=jnp.float32)

    @pl.when(k == 0)
    def _():
        acc_ref[:] = part

    @pl.when(k == 1)
    def _():
        logits = acc_ref[:] + part
        s = jax.nn.sigmoid(logits)
        sc = s + b_ref[:]

        gcols = []
        for g in range(N_GROUPS):
            vals = sc[g * GROUP_SIZE:(g + 1) * GROUP_SIZE, :]
            m1 = jnp.max(vals, axis=0, keepdims=True)
            eq = vals == m1
            cnt = jnp.sum(eq.astype(jnp.float32), axis=0, keepdims=True)
            m2 = jnp.max(jnp.where(eq, NEG, vals), axis=0, keepdims=True)
            gcols.append(m1 + jnp.where(cnt >= 2.0, m1, m2))
        gs = jnp.concatenate(gcols, axis=0)

        grow = jax.lax.broadcasted_iota(jnp.int32, (N_GROUPS, T), 0)
        sel_groups = jnp.zeros((N_GROUPS, T), jnp.bool_)
        gtmp = gs
        for _ in range(TOPK_GROUP):
            gm = jnp.max(gtmp, axis=0, keepdims=True)
            gi = jnp.min(jnp.where(gtmp == gm, grow, N_GROUPS), axis=0,
                         keepdims=True)
            hit = grow == gi
            sel_groups = sel_groups | hit
            gtmp = jnp.where(hit, NEG, gtmp)

        allowed = jnp.concatenate(
            [jnp.broadcast_to(sel_groups[g:g + 1, :], (GROUP_SIZE, T))
             for g in range(N_GROUPS)], axis=0)
        tmp = jnp.where(allowed, sc, 0.0)

        erow = jax.lax.broadcasted_iota(jnp.int32, (NUM_EXPERTS, T), 0)
        icols, wcols = [], []
        for _ in range(TOP_K):
            m = jnp.max(tmp, axis=0, keepdims=True)
            ei = jnp.min(jnp.where(tmp == m, erow, NUM_EXPERTS), axis=0,
                         keepdims=True)
            sel = erow == ei
            w = jnp.max(jnp.where(sel, s, NEG), axis=0, keepdims=True)
            icols.append(ei)
            wcols.append(w)
            tmp = jnp.where(sel, NEG, tmp)
        topk_idx = jnp.concatenate(icols, axis=0)
        topk_w = jnp.concatenate(wcols, axis=0)

        denom = jnp.sum(topk_w, axis=0, keepdims=True) + 1e-20
        topk_w = topk_w / denom * ROUTED_SCALING_FACTOR

        idx_ref[:] = topk_idx
        wt_ref[:] = topk_w


def kernel(x, W_gate, e_score_correction_bias):
    n = x.shape[0]
    T = 2048
    DHALF = DIM // 2
    b2 = e_score_correction_bias.reshape(NUM_EXPERTS, 1)
    idx_t, wt_t = pl.pallas_call(
        functools.partial(_router_body, T),
        grid=(n // T, 2),
        in_specs=[
            pl.BlockSpec((NUM_EXPERTS, DHALF), lambda i, k: (0, k)),
            pl.BlockSpec((T, DHALF), lambda i, k: (i, k)),
            pl.BlockSpec((NUM_EXPERTS, 1), lambda i, k: (0, 0)),
        ],
        out_specs=[
            pl.BlockSpec((TOP_K, T), lambda i, k: (0, i)),
            pl.BlockSpec((TOP_K, T), lambda i, k: (0, i)),
        ],
        out_shape=[
            jax.ShapeDtypeStruct((TOP_K, n), jnp.int32),
            jax.ShapeDtypeStruct((TOP_K, n), jnp.float32),
        ],
        scratch_shapes=[pltpu.VMEM((NUM_EXPERTS, T), jnp.float32)],
    )(W_gate, x, b2)
    return idx_t.T, wt_t.T


# speedup vs baseline: 5.2289x; 1.3417x over previous
import functools

import jax
import jax.numpy as jnp
from jax.experimental import pallas as pl

DIM = 2048
NUM_EXPERTS = 64
TOP_K = 8
N_GROUPS = 8
GROUP_SIZE = NUM_EXPERTS // N_GROUPS
TOPK_GROUP = 4
ROUTED_SCALING_FACTOR = 2.5

NEG = -1e30


def _router_body(T, w_ref, x_ref, b_ref, idx_ref, wt_ref):
    logits = jax.lax.dot_general(
        w_ref[:], x_ref[:], (((1,), (1,)), ((), ())),
        preferred_element_type=jnp.float32)
    s = jax.nn.sigmoid(logits)
    sc = s + b_ref[:]

    gcols = []
    for g in range(N_GROUPS):
        vals = sc[g * GROUP_SIZE:(g + 1) * GROUP_SIZE, :]
        m1 = jnp.max(vals, axis=0, keepdims=True)
        eq = vals == m1
        cnt = jnp.sum(eq.astype(jnp.float32), axis=0, keepdims=True)
        m2 = jnp.max(jnp.where(eq, NEG, vals), axis=0, keepdims=True)
        gcols.append(m1 + jnp.where(cnt >= 2.0, m1, m2))
    gs = jnp.concatenate(gcols, axis=0)

    grow = jax.lax.broadcasted_iota(jnp.int32, (N_GROUPS, T), 0)
    sel_groups = jnp.zeros((N_GROUPS, T), jnp.bool_)
    gtmp = gs
    for _ in range(TOPK_GROUP):
        gm = jnp.max(gtmp, axis=0, keepdims=True)
        gi = jnp.min(jnp.where(gtmp == gm, grow, N_GROUPS), axis=0,
                     keepdims=True)
        hit = grow == gi
        sel_groups = sel_groups | hit
        gtmp = jnp.where(hit, NEG, gtmp)

    allowed = jnp.concatenate(
        [jnp.broadcast_to(sel_groups[g:g + 1, :], (GROUP_SIZE, T))
         for g in range(N_GROUPS)], axis=0)
    tmp = jnp.where(allowed, sc, 0.0)

    erow = jax.lax.broadcasted_iota(jnp.int32, (NUM_EXPERTS, T), 0)
    icols, wcols = [], []
    for _ in range(TOP_K):
        m = jnp.max(tmp, axis=0, keepdims=True)
        ei = jnp.min(jnp.where(tmp == m, erow, NUM_EXPERTS), axis=0,
                     keepdims=True)
        sel = erow == ei
        w = jnp.max(jnp.where(sel, s, NEG), axis=0, keepdims=True)
        icols.append(ei)
        wcols.append(w)
        tmp = jnp.where(sel, NEG, tmp)
    topk_idx = jnp.concatenate(icols, axis=0)
    topk_w = jnp.concatenate(wcols, axis=0)

    denom = jnp.sum(topk_w, axis=0, keepdims=True) + 1e-20
    topk_w = topk_w / denom * ROUTED_SCALING_FACTOR

    idx_ref[:] = topk_idx
    wt_ref[:] = topk_w


def kernel(x, W_gate, e_score_correction_bias):
    n = x.shape[0]
    T = 2048
    grid = n // T
    b2 = e_score_correction_bias.reshape(NUM_EXPERTS, 1)
    idx_t, wt_t = pl.pallas_call(
        functools.partial(_router_body, T),
        grid=(grid,),
        in_specs=[
            pl.BlockSpec((NUM_EXPERTS, DIM), lambda i: (0, 0)),
            pl.BlockSpec((T, DIM), lambda i: (i, 0)),
            pl.BlockSpec((NUM_EXPERTS, 1), lambda i: (0, 0)),
        ],
        out_specs=[
            pl.BlockSpec((TOP_K, T), lambda i: (0, i)),
            pl.BlockSpec((TOP_K, T), lambda i: (0, i)),
        ],
        out_shape=[
            jax.ShapeDtypeStruct((TOP_K, n), jnp.int32),
            jax.ShapeDtypeStruct((TOP_K, n), jnp.float32),
        ],
    )(W_gate, x, b2)
    return idx_t.T, wt_t.T
